# Initial kernel scaffold; baseline (speedup 1.0000x reference)
import jax
import jax.numpy as jnp
from jax.experimental import pallas as pl


def kernel(x, edge_index, W1, a_src1, a_dst1, b1, W2, a_src2, a_dst2, b2, g1, be1, rm1, rv1, g2, be2, rm2, rv2, Wf1, bf1, Wf2, bf2):
    raise NotImplementedError("write your pallas kernel here")


# speedup vs baseline: 38.4755x; 38.4755x over previous
import functools

import jax
import jax.numpy as jnp
from jax import lax
from jax.experimental import pallas as pl
from jax.experimental.pallas import tpu as pltpu
from jax.experimental.pallas import tpu_sc as plsc

N = 10000
E = 320000
F = 128
NC, NS, L = 2, 16, 16
NW = NC * NS
EPW = E // NW
C = 80
NCHUNK = EPW // C
ROW1 = 144
ROW2 = 48
NP = 10112
RPT = NP // NS

_mesh = plsc.VectorSubcoreMesh(core_axis_name="c", subcore_axis_name="s")


def _tc_a_body(x_ref, w1_ref, ys_ref, yd_ref, h_ref, asn_ref, adn_ref):
    h = jnp.dot(x_ref[...], w1_ref[...], preferred_element_type=jnp.float32)
    h_ref[...] = h
    asn_ref[...] = jnp.dot(h, ys_ref[...], preferred_element_type=jnp.float32)
    adn_ref[...] = jnp.dot(h, yd_ref[...], preferred_element_type=jnp.float32)


def _tc_a(x, w1, ys, yd):
    blk = 1000
    return pl.pallas_call(
        _tc_a_body,
        grid=(N // blk,),
        in_specs=[
            pl.BlockSpec((blk, F), lambda i: (i, 0)),
            pl.BlockSpec((F, F), lambda i: (0, 0)),
            pl.BlockSpec((F, 16), lambda i: (0, 0)),
            pl.BlockSpec((F, 16), lambda i: (0, 0)),
        ],
        out_specs=[
            pl.BlockSpec((blk, F), lambda i: (i, 0)),
            pl.BlockSpec((blk, 16), lambda i: (i, 0)),
            pl.BlockSpec((blk, 16), lambda i: (i, 0)),
        ],
        out_shape=[
            jax.ShapeDtypeStruct((N, F), jnp.float32),
            jax.ShapeDtypeStruct((N, 16), jnp.float32),
            jax.ShapeDtypeStruct((N, 16), jnp.float32),
        ],
    )(x, w1, ys, yd)


@functools.partial(
    pl.kernel,
    mesh=_mesh,
    compiler_params=pltpu.CompilerParams(use_tc_tiling_on_sc=False, needs_layout_passes=False),
    out_type=jax.ShapeDtypeStruct((NC, NP, ROW1), jnp.float32),
    scratch_types=[
        pltpu.VMEM((C,), jnp.int32),
        pltpu.VMEM((C,), jnp.int32),
        pltpu.VMEM((C, 16), jnp.float32),
        pltpu.VMEM((C, 16), jnp.float32),
        pltpu.VMEM((C, F), jnp.float32),
        pltpu.VMEM((C, ROW1), jnp.float32),
        pltpu.VMEM_SHARED((NP, ROW1), jnp.float32),
        pltpu.SemaphoreType.DMA,
    ],
)
def _sc_edge1(src_hbm, dst_hbm, asn_hbm, adn_hbm, h_hbm, zz_hbm, out_hbm,
              src_v, dst_v, as_v, ad_v, h_v, row_v, acc_sh, sem):
    cid = lax.axis_index("c")
    sid = lax.axis_index("s")
    wid = cid * NS + sid

    pltpu.sync_copy(zz_hbm, acc_sh.at[pl.ds(sid * RPT, RPT)])
    plsc.subcore_barrier()

    mask8 = jnp.where(lax.iota(jnp.int32, L) < 8, 1.0, 0.0)

    def chunk(j, _):
        base = wid * EPW + j * C
        pltpu.sync_copy(src_hbm.at[pl.ds(base, C)], src_v)
        pltpu.sync_copy(dst_hbm.at[pl.ds(base, C)], dst_v)
        hcp = pltpu.async_copy(h_hbm.at[src_v], h_v, sem)
        pltpu.sync_copy(asn_hbm.at[src_v], as_v)
        pltpu.sync_copy(adn_hbm.at[dst_v], ad_v)

        def wbody(c, _):
            s = as_v[c, :] + ad_v[c, :]
            e = jnp.where(s >= 0.0, s, 0.2 * s)
            row_v[c, pl.ds(F, L)] = jnp.exp(e) * mask8
            return _

        lax.fori_loop(0, C, wbody, None, unroll=4)
        hcp.wait()

        def sbody(c, _):
            wv = row_v[c, pl.ds(F, L)]
            for k in range(8):
                w_s = wv[k]
                row_v[c, pl.ds(16 * k, 16)] = h_v[c, pl.ds(16 * k, 16)] * w_s
            return _

        lax.fori_loop(0, C, sbody, None)
        pltpu.sync_copy(row_v, acc_sh.at[dst_v], add=True)
        return _

    lax.fori_loop(0, NCHUNK, chunk, None)
    plsc.subcore_barrier()
    pltpu.sync_copy(acc_sh.at[pl.ds(sid * RPT, RPT)],
                    out_hbm.at[cid, pl.ds(sid * RPT, RPT)])


def _tc_b_body(acc_ref, p_ref, b1_ref, sc1_ref, sh1_ref, w2_ref, c2_ref,
               h2p_ref, a2_ref):
    a = acc_ref[0] + acc_ref[1]
    num = a[:, :F]
    den = a[:, F:F + 8]
    denf = jnp.dot(den, p_ref[...], preferred_element_type=jnp.float32)
    h1 = jax.nn.relu(num / (denf + 1e-16) + b1_ref[...])
    hb = h1 * sc1_ref[...] + sh1_ref[...]
    h2p = jnp.dot(hb, w2_ref[...], preferred_element_type=jnp.float32)
    h2p_ref[...] = h2p
    a2_ref[...] = jnp.dot(h2p, c2_ref[...], preferred_element_type=jnp.float32)


def _tc_b(acc1, p, b1, sc1, sh1, w2, c2):
    blk = 1000
    return pl.pallas_call(
        _tc_b_body,
        grid=(N // blk,),
        in_specs=[
            pl.BlockSpec((NC, blk, ROW1), lambda i: (0, i, 0)),
            pl.BlockSpec((8, F), lambda i: (0, 0)),
            pl.BlockSpec((1, F), lambda i: (0, 0)),
            pl.BlockSpec((1, F), lambda i: (0, 0)),
            pl.BlockSpec((1, F), lambda i: (0, 0)),
            pl.BlockSpec((F, 32), lambda i: (0, 0)),
            pl.BlockSpec((32, 8), lambda i: (0, 0)),
        ],
        out_specs=[
            pl.BlockSpec((blk, 32), lambda i: (i, 0)),
            pl.BlockSpec((blk, 8), lambda i: (i, 0)),
        ],
        out_shape=[
            jax.ShapeDtypeStruct((N, 32), jnp.float32),
            jax.ShapeDtypeStruct((N, 8), jnp.float32),
        ],
    )(acc1, p, b1, sc1, sh1, w2, c2)


@functools.partial(
    pl.kernel,
    mesh=_mesh,
    compiler_params=pltpu.CompilerParams(use_tc_tiling_on_sc=False, needs_layout_passes=False),
    out_type=jax.ShapeDtypeStruct((NC, NP, ROW2), jnp.float32),
    scratch_types=[
        pltpu.VMEM((C,), jnp.int32),
        pltpu.VMEM((C,), jnp.int32),
        pltpu.VMEM((N, 8), jnp.float32),
        pltpu.VMEM((C,), jnp.float32),
        pltpu.VMEM((C, 32), jnp.float32),
        pltpu.VMEM((C, ROW2), jnp.float32),
        pltpu.VMEM_SHARED((NP, ROW2), jnp.float32),
        pltpu.SemaphoreType.DMA,
    ],
)
def _sc_edge2(src_hbm, dst_hbm, a2_hbm, h2_hbm, zz_hbm, out_hbm,
              src_v, dst_v, a2_v, w_v, h_v, row_v, acc_sh, sem):
    cid = lax.axis_index("c")
    sid = lax.axis_index("s")
    wid = cid * NS + sid

    pltpu.sync_copy(zz_hbm, acc_sh.at[pl.ds(sid * RPT, RPT)])
    pltpu.sync_copy(a2_hbm, a2_v)
    plsc.subcore_barrier()

    lane = lax.iota(jnp.int32, L)
    zero16 = jnp.zeros((L,), jnp.int32)
    one16 = jnp.ones((L,), jnp.int32)
    onehot0 = jnp.where(lane == 0, 1.0, 0.0)

    def chunk(j, _):
        base = wid * EPW + j * C
        pltpu.sync_copy(src_hbm.at[pl.ds(base, C)], src_v)
        pltpu.sync_copy(dst_hbm.at[pl.ds(base, C)], dst_v)
        hcp = pltpu.async_copy(h2_hbm.at[src_v], h_v, sem)

        def wbody(t, _):
            s16 = src_v[pl.ds(t * L, L)]
            d16 = dst_v[pl.ds(t * L, L)]
            asv = plsc.load_gather(a2_v, [s16, zero16])
            adv = plsc.load_gather(a2_v, [d16, one16])
            s = asv + adv
            e = jnp.where(s >= 0.0, s, 0.2 * s)
            w_v[pl.ds(t * L, L)] = jnp.exp(e)
            return _

        lax.fori_loop(0, C // L, wbody, None)
        hcp.wait()

        def sbody(t, _):
            wv = w_v[pl.ds(t * L, L)]
            for l in range(L):
                c = t * L + l
                w_s = wv[l]
                row_v[c, pl.ds(0, 16)] = h_v[c, pl.ds(0, 16)] * w_s
                row_v[c, pl.ds(16, 16)] = h_v[c, pl.ds(16, 16)] * w_s
                row_v[c, pl.ds(32, 16)] = onehot0 * w_s
            return _

        lax.fori_loop(0, C // L, sbody, None)
        pltpu.sync_copy(row_v, acc_sh.at[dst_v], add=True)
        return _

    lax.fori_loop(0, NCHUNK, chunk, None)
    plsc.subcore_barrier()
    pltpu.sync_copy(acc_sh.at[pl.ds(sid * RPT, RPT)],
                    out_hbm.at[cid, pl.ds(sid * RPT, RPT)])


def _tc_c_body(acc_ref, b2_ref, sc2_ref, sh2_ref, wf1_ref, bf1_ref,
               wf2_ref, bf2_ref, out_ref):
    a = acc_ref[0] + acc_ref[1]
    num = a[:, :32]
    den = a[:, 32:33]
    h2 = jax.nn.relu(num / (den + 1e-16) + b2_ref[...])
    hb = h2 * sc2_ref[...] + sh2_ref[...]
    f = jax.nn.relu(jnp.dot(hb, wf1_ref[...], preferred_element_type=jnp.float32)
                    + bf1_ref[...])
    o = jnp.dot(f, wf2_ref[...], preferred_element_type=jnp.float32) + bf2_ref[...]
    m = jnp.max(o, axis=1, keepdims=True)
    ls = jnp.log(jnp.sum(jnp.exp(o - m), axis=1, keepdims=True)) + m
    out_ref[...] = o - ls


def _tc_c(acc2, b2, sc2, sh2, wf1, bf1, wf2, bf2):
    blk = 1000
    nclass = wf2.shape[1]
    return pl.pallas_call(
        _tc_c_body,
        grid=(N // blk,),
        in_specs=[
            pl.BlockSpec((NC, blk, ROW2), lambda i: (0, i, 0)),
            pl.BlockSpec((1, 32), lambda i: (0, 0)),
            pl.BlockSpec((1, 32), lambda i: (0, 0)),
            pl.BlockSpec((1, 32), lambda i: (0, 0)),
            pl.BlockSpec((32, 32), lambda i: (0, 0)),
            pl.BlockSpec((1, 32), lambda i: (0, 0)),
            pl.BlockSpec((32, nclass), lambda i: (0, 0)),
            pl.BlockSpec((1, nclass), lambda i: (0, 0)),
        ],
        out_specs=pl.BlockSpec((blk, nclass), lambda i: (i, 0)),
        out_shape=jax.ShapeDtypeStruct((N, nclass), jnp.float32),
    )(acc2, b2, sc2, sh2, wf1, bf1, wf2, bf2)


def kernel(x, edge_index, W1, a_src1, a_dst1, b1, W2, a_src2, a_dst2, b2,
           g1, be1, rm1, rv1, g2, be2, rm2, rv2, Wf1, bf1, Wf2, bf2):
    eps = 1e-5
    src = edge_index[0].astype(jnp.int32)
    dst = edge_index[1].astype(jnp.int32)

    ys = jnp.zeros((F, 16), jnp.float32)
    ys = ys.at[:, :8].set(
        jax.scipy.linalg.block_diag(*[a_src1[k, :, None] for k in range(8)]))
    yd = jnp.zeros((F, 16), jnp.float32)
    yd = yd.at[:, :8].set(
        jax.scipy.linalg.block_diag(*[a_dst1[k, :, None] for k in range(8)]))

    p = jnp.repeat(jnp.eye(8, dtype=jnp.float32), 16, axis=1)

    sc1v = g1 / jnp.sqrt(rv1 + eps)
    sh1v = be1 - rm1 * sc1v
    sc2v = g2 / jnp.sqrt(rv2 + eps)
    sh2v = be2 - rm2 * sc2v
    c2 = jnp.stack([a_src2[0], a_dst2[0]], axis=1)
    c2 = jnp.pad(c2, ((0, 0), (0, 6)))

    h, asn, adn = _tc_a(x, W1, ys, yd)

    zz1 = jnp.zeros((RPT, ROW1), jnp.float32)
    acc1 = _sc_edge1(src, dst, asn, adn, h, zz1)

    h2p, a2 = _tc_b(acc1, p, b1.reshape(1, F), sc1v.reshape(1, F),
                    sh1v.reshape(1, F), W2, c2)

    zz2 = jnp.zeros((RPT, ROW2), jnp.float32)
    acc2 = _sc_edge2(src, dst, a2[:, :8], h2p, zz2)

    return _tc_c(acc2, b2.reshape(1, 32), sc2v.reshape(1, 32),
                 sh2v.reshape(1, 32), Wf1, bf1.reshape(1, 32),
                 Wf2, bf2.reshape(1, 40))
